# baseline (speedup 1.0000x reference)
import functools

import jax
import jax.numpy as jnp
from jax import lax
from jax.experimental import pallas as pl
from jax.experimental.pallas import tpu as pltpu
from jax.experimental.pallas import tpu_sc as plsc

VOCAB = 128
DIM = 128
SEQ = 512
BATCH = 1024

NC = 2
NS = 16
NW = NC * NS
LANES = 16

ROWS = BATCH * SEQ
RPW = ROWS // NW
BPT = BATCH // NW
HB = 8
SB = 32
NQ = BPT // HB
NJ = SEQ // SB
NBLK = NQ * NJ
NBUF = 2
KD = DIM // LANES


def _emb_body(xr_hbm, lw_hbm, pw_hbm, out_hbm,
              idx_v, l_v, pp0, pp1, buf0, buf1,
              q0, q1, s0, s1):
    wid = lax.axis_index("s") * NC + lax.axis_index("c")
    row0 = wid * RPW

    bufs = (buf0, buf1)
    pps = (pp0, pp1)
    psem = (q0, q1)
    ssem = (s0, s1)

    pltpu.sync_copy(xr_hbm.at[wid], idx_v)
    pltpu.sync_copy(lw_hbm, l_v)

    pltpu.async_copy(pw_hbm.at[pl.ds(0, SB)], pps[0], psem[0])

    def outer(i, carry):
        for par in range(NBUF):
            jb = i * NBUF + par
            q = jb % NQ
            pbase = (jb // NQ) * SB
            jn = jb + 1
            bn = (par + 1) % NBUF

            if par == 0:
                pltpu.async_copy(
                    pw_hbm.at[pl.ds((jn // NQ) * SB, SB)], pps[bn], psem[bn]
                )
            else:
                @pl.when(i < NBLK // NBUF - 1)
                def _():
                    pltpu.async_copy(
                        pw_hbm.at[pl.ds((jn // NQ) * SB, SB)],
                        pps[bn], psem[bn],
                    )

            @pl.when(i >= 1)
            def _():
                for _ in range(HB):
                    pltpu.make_async_copy(
                        bufs[par].at[pl.ds(0, SB)],
                        out_hbm.at[pl.ds(0, SB)],
                        ssem[par],
                    ).wait()
            pltpu.make_async_copy(
                pw_hbm.at[pl.ds(pbase, SB)], pps[par], psem[par]
            ).wait()

            def pair_body(p, carry2):
                toks = idx_v[jb, pl.ds(p * 2 * HB, 2 * HB)]
                prow0 = [
                    pps[par][2 * p, pl.ds(k * LANES, LANES)]
                    for k in range(KD)
                ]
                prow1 = [
                    pps[par][2 * p + 1, pl.ds(k * LANES, LANES)]
                    for k in range(KD)
                ]
                for bi in range(HB):
                    ta = toks[bi]
                    tb = toks[HB + bi]
                    la = [
                        l_v[ta, pl.ds(k * LANES, LANES)] for k in range(KD)
                    ]
                    lb = [
                        l_v[tb, pl.ds(k * LANES, LANES)] for k in range(KD)
                    ]
                    for k in range(KD):
                        sl = pl.ds(k * LANES, LANES)
                        bufs[par][bi * SB + 2 * p, sl] = la[k] + prow0[k]
                        bufs[par][bi * SB + 2 * p + 1, sl] = (
                            lb[k] + prow1[k]
                        )
                return carry2

            lax.fori_loop(0, SB // 2, pair_body, 0, unroll=False)

            for bi in range(HB):
                pltpu.async_copy(
                    bufs[par].at[pl.ds(bi * SB, SB)],
                    out_hbm.at[
                        pl.ds(row0 + (q * HB + bi) * SEQ + pbase, SB)
                    ],
                    ssem[par],
                )
        return carry

    lax.fori_loop(0, NBLK // NBUF, outer, 0, unroll=False)

    for par in range(NBUF):
        for _ in range(HB):
            pltpu.make_async_copy(
                bufs[par].at[pl.ds(0, SB)],
                out_hbm.at[pl.ds(0, SB)],
                ssem[par],
            ).wait()


_emb = functools.partial(
    pl.kernel,
    out_type=jax.ShapeDtypeStruct((ROWS, DIM), jnp.float32),
    mesh=plsc.VectorSubcoreMesh(core_axis_name="c", subcore_axis_name="s"),
    scratch_types=[
        pltpu.VMEM((NBLK, SB * HB), jnp.int32),
        pltpu.VMEM((VOCAB, DIM), jnp.float32),
        pltpu.VMEM((SB, DIM), jnp.float32),
        pltpu.VMEM((SB, DIM), jnp.float32),
        pltpu.VMEM((HB * SB, DIM), jnp.float32),
        pltpu.VMEM((HB * SB, DIM), jnp.float32),
        pltpu.SemaphoreType.DMA,
        pltpu.SemaphoreType.DMA,
        pltpu.SemaphoreType.DMA,
        pltpu.SemaphoreType.DMA,
    ],
)(_emb_body)


@jax.jit
def kernel(x, embedLettre_w, embedPosition_w):
    xr = (
        x.reshape(NW, NQ, HB, NJ, SB)
        .transpose(0, 3, 1, 4, 2)
        .reshape(NW, NBLK, SB * HB)
    )
    out = _emb(xr, embedLettre_w, embedPosition_w)
    return out.reshape(BATCH, SEQ, DIM)
